# baseline (speedup 1.0000x reference)
import jax
import jax.numpy as jnp
from jax.experimental import pallas as pl
from jax.experimental.pallas import tpu as pltpu
from jax.experimental.pallas import tpu_sc as plsc

_W = 256
_L = 16


def _gather_transposed(w_rm, idx4, s, b, d):
    nb = b // _W
    mesh = plsc.VectorSubcoreMesh(core_axis_name="core", subcore_axis_name="subcore")

    @pl.kernel(
        out_type=jax.ShapeDtypeStruct((s, d, b), w_rm.dtype),
        mesh=mesh,
        scratch_types=[pltpu.VMEM((_W, d), w_rm.dtype)],
        compiler_params=pltpu.CompilerParams(
            use_tc_tiling_on_sc=False, needs_layout_passes=False
        ),
    )
    def gather_kernel(w_hbm, i_hbm, o_hbm, scratch):
        def body(i_vmem, o_vmem):
            pltpu.sync_copy(w_hbm.at[i_vmem.at[0, 0, 0]], scratch)

            @pl.loop(0, d)
            def _(dd):
                @pl.loop(0, _W, step=_L)
                def _(r0):
                    rows = r0 + jax.lax.iota(jnp.int32, _L)
                    cols = jnp.full((_L,), dd, jnp.int32)
                    o_vmem[0, dd, pl.ds(r0, _L)] = plsc.load_gather(
                        scratch, [rows, cols]
                    )

        pltpu.emit_pipeline(
            body,
            grid=(s, nb),
            in_specs=[
                pl.BlockSpec((1, 1, 1, _W), index_map=lambda i, j: (i, j, 0, 0))
            ],
            out_specs=[pl.BlockSpec((1, d, _W), index_map=lambda i, j: (i, 0, j))],
            core_axis_name=("core", "subcore"),
            dimension_semantics=(pltpu.PARALLEL, pltpu.PARALLEL),
        )(i_hbm, o_hbm)

    return gather_kernel(w_rm, idx4)


def kernel(x, weight):
    b, s = x.shape
    d = weight.shape[1]
    idx4 = x.T.reshape(s, b // _W, 1, _W).astype(jnp.int32)
    out_p = _gather_transposed(weight, idx4, s, b, d)
    return jnp.transpose(out_p, (2, 0, 1))


# speedup vs baseline: 1.0443x; 1.0443x over previous
import jax
import jax.numpy as jnp
from jax.experimental import pallas as pl
from jax.experimental.pallas import tpu as pltpu
from jax.experimental.pallas import tpu_sc as plsc

_W = 256
_L = 16


def _gather_transposed(w_rm, idx4, s, b, d):
    nb = b // _W
    mesh = plsc.VectorSubcoreMesh(core_axis_name="core", subcore_axis_name="subcore")

    @pl.kernel(
        out_type=jax.ShapeDtypeStruct((s, d, b), w_rm.dtype),
        mesh=mesh,
        scratch_types=[pltpu.VMEM((_W, d), w_rm.dtype)],
        compiler_params=pltpu.CompilerParams(
            use_tc_tiling_on_sc=False, needs_layout_passes=False
        ),
    )
    def gather_kernel(w_hbm, i_hbm, o_hbm, scratch):
        def body(i_vmem, o_vmem):
            pltpu.sync_copy(w_hbm.at[i_vmem.at[0, 0, 0]], scratch)
            base = jax.lax.iota(jnp.int32, _L)
            rows = [base + r0 for r0 in range(0, _W, _L)]

            @pl.loop(0, d)
            def _(dd):
                cols = jnp.full((_L,), dd, jnp.int32)
                for k, r0 in enumerate(range(0, _W, _L)):
                    o_vmem[0, dd, pl.ds(r0, _L)] = plsc.load_gather(
                        scratch, [rows[k], cols]
                    )

        pltpu.emit_pipeline(
            body,
            grid=(s, nb),
            in_specs=[
                pl.BlockSpec((1, 1, 1, _W), index_map=lambda i, j: (i, j, 0, 0))
            ],
            out_specs=[pl.BlockSpec((1, d, _W), index_map=lambda i, j: (i, 0, j))],
            core_axis_name=("core", "subcore"),
            dimension_semantics=(pltpu.PARALLEL, pltpu.PARALLEL),
        )(i_hbm, o_hbm)

    return gather_kernel(w_rm, idx4)


def kernel(x, weight):
    b, s = x.shape
    d = weight.shape[1]
    idx4 = x.T.reshape(s, b // _W, 1, _W).astype(jnp.int32)
    out_p = _gather_transposed(weight, idx4, s, b, d)
    return jnp.transpose(out_p, (2, 0, 1))


# speedup vs baseline: 1.1457x; 1.0971x over previous
import jax
import jax.numpy as jnp
from jax.experimental import pallas as pl
from jax.experimental.pallas import tpu as pltpu
from jax.experimental.pallas import tpu_sc as plsc

_W = 256
_L = 16
_TD = 8
_TB = 128


def _gather_tiled(w_rm, idx4, s, b, d):
    nb = b // _W
    ndg = d // _TD
    nbt = b // _TB
    bt_per_w = _W // _TB
    mesh = plsc.VectorSubcoreMesh(core_axis_name="core", subcore_axis_name="subcore")

    @pl.kernel(
        out_type=jax.ShapeDtypeStruct((s, ndg, nbt, _TD, _TB), w_rm.dtype),
        mesh=mesh,
        scratch_types=[pltpu.VMEM((_W, d), w_rm.dtype)],
        compiler_params=pltpu.CompilerParams(
            use_tc_tiling_on_sc=False, needs_layout_passes=False
        ),
    )
    def gather_kernel(w_hbm, i_hbm, o_hbm, scratch):
        def body(i_vmem, o_vmem):
            pltpu.sync_copy(w_hbm.at[i_vmem.at[0, 0, 0]], scratch)
            base = jax.lax.iota(jnp.int32, _L)
            rows = [
                [base + (bt * _TB + c * _L) for c in range(_TB // _L)]
                for bt in range(bt_per_w)
            ]

            @pl.loop(0, d)
            def _(dd):
                dg = dd // _TD
                dr = dd % _TD
                cols = jnp.full((_L,), dd, jnp.int32)
                for bt in range(bt_per_w):
                    for c in range(_TB // _L):
                        o_vmem[0, dg, bt, dr, pl.ds(c * _L, _L)] = plsc.load_gather(
                            scratch, [rows[bt][c], cols]
                        )

        pltpu.emit_pipeline(
            body,
            grid=(s, nb),
            in_specs=[
                pl.BlockSpec((1, 1, 1, _W), index_map=lambda i, j: (i, j, 0, 0))
            ],
            out_specs=[
                pl.BlockSpec(
                    (1, ndg, bt_per_w, _TD, _TB),
                    index_map=lambda i, j: (i, 0, j, 0, 0),
                )
            ],
            core_axis_name=("core", "subcore"),
            dimension_semantics=(pltpu.PARALLEL, pltpu.PARALLEL),
        )(i_hbm, o_hbm)

    return gather_kernel(w_rm, idx4)


def kernel(x, weight):
    b, s = x.shape
    d = weight.shape[1]
    idx4 = x.T.reshape(s, b // _W, 1, _W).astype(jnp.int32)
    out5 = _gather_tiled(weight, idx4, s, b, d)
    return jnp.transpose(out5, (2, 4, 0, 1, 3)).reshape(b, s, d)
